# baseline (speedup 1.0000x reference)
import functools

import jax
import jax.numpy as jnp
from jax import lax
from jax.experimental import pallas as pl
from jax.experimental.pallas import tpu as pltpu
from jax.experimental.pallas import tpu_sc as plsc


def _tc_body(src_ref, dis_ref, w1_ref, b1_ref, w2_ref, b2_ref, ratio_ref,
             table_ref, idx_ref, loss_ref):
    b = pl.program_id(0)
    x = src_ref[0]
    dis = dis_ref[0]
    xd = x * dis
    hid = lax.dot_general(w1_ref[...], xd, (((1,), (0,)), ((), ())),
                          preferred_element_type=jnp.float32)
    hid = jax.nn.relu(hid + b1_ref[...])
    scores = lax.dot_general(w2_ref[...], hid, (((1,), (0,)), ((), ())),
                             preferred_element_type=jnp.float32)
    scores = scores + b2_ref[0, 0]
    sw_row = jax.nn.sigmoid(scores) * ratio_ref[0, 0]

    mu = jnp.mean(x, axis=0, keepdims=True)
    var = jnp.mean((x - mu) ** 2, axis=0, keepdims=True)
    srcn = (x - mu) * lax.rsqrt(var + 1e-5)

    table_ref[0] = jnp.transpose(srcn * sw_row)

    hw = sw_row.shape[1]
    sw_col = jnp.transpose(sw_row)
    jj = lax.broadcasted_iota(jnp.int32, (hw, hw), 0)
    ii = lax.broadcasted_iota(jnp.int32, (hw, hw), 1)
    beats = (sw_col > sw_row) | ((sw_col == sw_row) & (jj < ii))
    rank_row = jnp.sum(jnp.where(beats, 1, 0), axis=0, keepdims=True)
    rank_col = jnp.transpose(rank_row)
    hits = rank_col == ii
    idx_row = jnp.sum(jnp.where(hits, jj, 0), axis=0, keepdims=True)
    idx_ref[0] = idx_row

    partial = jnp.sum(sw_row) / (32.0 * hw)
    prev = jnp.where(b == 0, 0.0, loss_ref[0, 0])
    loss_ref[0, 0] = prev + partial


def _tc_stage(src3, dis3, w1, b1c, w2, b2s, ratio):
    bs, c, hw = src3.shape
    return pl.pallas_call(
        _tc_body,
        grid=(bs,),
        in_specs=[
            pl.BlockSpec((1, c, hw), lambda b: (b, 0, 0)),
            pl.BlockSpec((1, 1, hw), lambda b: (b, 0, 0)),
            pl.BlockSpec((c, c), lambda b: (0, 0)),
            pl.BlockSpec((c, 1), lambda b: (0, 0)),
            pl.BlockSpec((1, c), lambda b: (0, 0)),
            pl.BlockSpec(memory_space=pltpu.SMEM),
            pl.BlockSpec(memory_space=pltpu.SMEM),
        ],
        out_specs=[
            pl.BlockSpec((1, hw, c), lambda b: (b, 0, 0)),
            pl.BlockSpec((1, 1, hw), lambda b: (b, 0, 0)),
            pl.BlockSpec(memory_space=pltpu.SMEM),
        ],
        out_shape=[
            jax.ShapeDtypeStruct((bs, hw, c), jnp.float32),
            jax.ShapeDtypeStruct((bs, 1, hw), jnp.int32),
            jax.ShapeDtypeStruct((1, 1), jnp.float32),
        ],
    )(src3, dis3, w1, b1c, w2, b2s, ratio)


def _sc_stage(table_flat, idxt_flat, pe_flat, bs, c, hw):
    info = plsc.get_sparse_core_info()
    nc, ns = info.num_cores, info.num_subcores
    nw = nc * ns
    rows_per_w = (bs * hw) // nw
    chunk = 256

    mesh = plsc.VectorSubcoreMesh(core_axis_name="c", subcore_axis_name="s")

    @functools.partial(
        pl.kernel, mesh=mesh,
        out_type=[
            jax.ShapeDtypeStruct((bs * hw, c), jnp.float32),
            jax.ShapeDtypeStruct((bs * hw,), jnp.float32),
        ],
        scratch_types=[
            pltpu.VMEM((rows_per_w,), jnp.int32),
            pltpu.VMEM((rows_per_w,), jnp.int32),
            pltpu.VMEM((rows_per_w,), jnp.int32),
            pltpu.VMEM((chunk, c), jnp.float32),
            pltpu.VMEM((rows_per_w,), jnp.float32),
            pltpu.SemaphoreType.DMA,
        ],
    )
    def run(table_hbm, idxt_hbm, pe_hbm, out_hbm, outpe_hbm,
            idx_v, rowidx_v, peidx_v, rows_v, peout_v, sem):
        wid = lax.axis_index("s") * nc + lax.axis_index("c")
        base = wid * rows_per_w
        pltpu.sync_copy(idxt_hbm.at[pl.ds(base, rows_per_w)], idx_v)
        for j in range(rows_per_w // 16):
            sl = pl.ds(j * 16, 16)
            v = idx_v[sl]
            r = base + j * 16 + lax.broadcasted_iota(jnp.int32, (16,), 0)
            bvec = lax.bitwise_and(r, bs - 1)
            rowidx_v[sl] = v + bvec * hw
            peidx_v[sl] = v * (bs * c) + bvec * c
        for k in range(rows_per_w // chunk):
            pltpu.async_copy(
                table_hbm.at[rowidx_v.at[pl.ds(k * chunk, chunk)]],
                rows_v, sem).wait()
            pltpu.sync_copy(rows_v, out_hbm.at[pl.ds(base + k * chunk, chunk)])
        pltpu.async_copy(pe_hbm.at[peidx_v], peout_v, sem).wait()
        pltpu.sync_copy(peout_v, outpe_hbm.at[pl.ds(base, rows_per_w)])

    return run(table_flat, idxt_flat, pe_flat)


def kernel(src, pos_embed, sample_ratio, dis_priority, W1, b1, W2, b2):
    bs, c, h, w = src.shape
    hw = h * w
    src3 = src.reshape(bs, c, hw)
    dis3 = dis_priority.reshape(bs, 1, hw)
    b1c = b1.reshape(c, 1)
    b2s = b2.reshape(1, 1)
    ratio = jnp.asarray(sample_ratio, jnp.float32).reshape(1, 1)

    table, idx3, loss = _tc_stage(src3, dis3, W1, b1c, W2, b2s, ratio)
    idx = idx3.reshape(bs, hw)
    idxt_flat = jnp.transpose(idx).reshape(-1)

    out_flat, out_pe = _sc_stage(
        table.reshape(bs * hw, c), idxt_flat, pos_embed.reshape(-1),
        bs, c, hw)

    return (out_flat.reshape(hw, bs, c), loss.reshape(()), idx,
            out_pe.reshape(hw, bs, 1))
